# baseline (speedup 1.0000x reference)
import jax
import jax.numpy as jnp
from jax import lax
from jax.experimental import pallas as pl
from jax.experimental.pallas import tpu as pltpu

B = 16
D = 64
K_DB = 1_000_000
K_TOP = 10
S = 8192
G = (K_DB + S - 1) // S
CAND = 128
FEAT_IN = 150528
FEAT_CHUNK = 7168
FEAT_G = FEAT_IN // FEAT_CHUNK
IMAX = jnp.iinfo(jnp.int32).max


def _feat_kernel(x_ref, w_ref, o_ref):
    @pl.when(pl.program_id(0) == 0)
    def _():
        o_ref[...] = jnp.zeros_like(o_ref)

    o_ref[...] += jnp.dot(x_ref[...], w_ref[...],
                          preferred_element_type=jnp.float32)


def _extract_topk(v, i, n):
    outv, outi = [], []
    for _ in range(n):
        m = jnp.max(v, axis=1, keepdims=True)
        am = jnp.min(jnp.where(v == m, i, IMAX), axis=1, keepdims=True)
        outv.append(m)
        outi.append(am)
        v = jnp.where((v == m) & (i == am), -jnp.inf, v)
    return jnp.concatenate(outv, axis=1), jnp.concatenate(outi, axis=1)


def _topk_kernel(feat_ref, db_ref, vals_ref, idx_ref, cv_ref, ci_ref):
    g = pl.program_id(0)
    s = lax.dot_general(feat_ref[...], db_ref[...], (((1,), (1,)), ((), ())),
                        preferred_element_type=jnp.float32)
    gidx = lax.broadcasted_iota(jnp.int32, (B, S), 1) + g * S
    s = jnp.where(gidx < K_DB, s, -jnp.inf)

    cv, ci = _extract_topk(s, gidx, K_TOP)
    pad_v = jnp.full((B, CAND - K_TOP), -jnp.inf, jnp.float32)
    pad_i = jnp.full((B, CAND - K_TOP), IMAX, jnp.int32)
    cv_ref[:, pl.ds(g * CAND, CAND)] = jnp.concatenate([cv, pad_v], axis=1)
    ci_ref[:, pl.ds(g * CAND, CAND)] = jnp.concatenate([ci, pad_i], axis=1)

    @pl.when(g == G - 1)
    def _():
        fv, fi = _extract_topk(cv_ref[...], ci_ref[...], K_TOP)
        vals_ref[...] = fv
        idx_ref[...] = fi


def kernel(image, k, W, database):
    x = image.reshape(B, FEAT_IN)
    feat = pl.pallas_call(
        _feat_kernel,
        grid=(FEAT_G,),
        in_specs=[
            pl.BlockSpec((B, FEAT_CHUNK), lambda g: (0, g)),
            pl.BlockSpec((FEAT_CHUNK, D), lambda g: (g, 0)),
        ],
        out_specs=pl.BlockSpec((B, D), lambda g: (0, 0)),
        out_shape=jax.ShapeDtypeStruct((B, D), jnp.float32),
        compiler_params=pltpu.CompilerParams(
            dimension_semantics=("arbitrary",)),
    )(x, W)

    vals, idx = pl.pallas_call(
        _topk_kernel,
        grid=(G,),
        in_specs=[
            pl.BlockSpec((B, D), lambda g: (0, 0)),
            pl.BlockSpec((S, D), lambda g: (g, 0)),
        ],
        out_specs=[
            pl.BlockSpec((B, K_TOP), lambda g: (0, 0)),
            pl.BlockSpec((B, K_TOP), lambda g: (0, 0)),
        ],
        out_shape=[
            jax.ShapeDtypeStruct((B, K_TOP), jnp.float32),
            jax.ShapeDtypeStruct((B, K_TOP), jnp.int32),
        ],
        scratch_shapes=[
            pltpu.VMEM((B, G * CAND), jnp.float32),
            pltpu.VMEM((B, G * CAND), jnp.int32),
        ],
        compiler_params=pltpu.CompilerParams(
            dimension_semantics=("arbitrary",)),
    )(feat, database)

    return vals, idx
